# Initial kernel scaffold; baseline (speedup 1.0000x reference)
import jax
import jax.numpy as jnp
from jax.experimental import pallas as pl


def kernel(x, edge_index, pair_index, W_enc, b_enc, W_lin, att_src, att_dst, bias_gat, W1, b1, W2, b2):
    raise NotImplementedError("write your pallas kernel here")


# speedup vs baseline: 1.0467x; 1.0467x over previous
import jax
import jax.numpy as jnp
from jax.experimental import pallas as pl

N = 10000
E = 640000
D_FEAT = 128
HID = 32
HEADS = 2
P = 200000


def _encode_body(x_ref, wenc_ref, benc_ref, wlin_ref, acat_ref, xh_ref, a4_ref):
    h = jnp.maximum(
        jnp.dot(x_ref[...], wenc_ref[...], preferred_element_type=jnp.float32)
        + benc_ref[0:1, :],
        0.0,
    )
    xh = jnp.dot(h, wlin_ref[...], preferred_element_type=jnp.float32)
    xh_ref[...] = xh
    ps = xh * acat_ref[0:1, :]
    pd = xh * acat_ref[1:2, :]
    a_s0 = jnp.sum(ps[:, :HID], axis=1, keepdims=True)
    a_s1 = jnp.sum(ps[:, HID:], axis=1, keepdims=True)
    a_d0 = jnp.sum(pd[:, :HID], axis=1, keepdims=True)
    a_d1 = jnp.sum(pd[:, HID:], axis=1, keepdims=True)
    t0 = a_s0 + a_d0
    t1 = a_s1 + a_d1
    w0 = jnp.exp(jnp.maximum(t0, 0.2 * t0))
    w1 = jnp.exp(jnp.maximum(t1, 0.2 * t1))
    a4_ref[...] = jnp.concatenate(
        [a_s0, a_s1, a_d0, a_d1, w0, w1, t0, t1], axis=1
    )


def _encode(x, W_enc, b_enc, W_lin, att_src, att_dst):
    benc2 = jnp.tile(b_enc[None, :], (8, 1))
    acat = jnp.concatenate(
        [att_src.reshape(1, HEADS * HID), att_dst.reshape(1, HEADS * HID)], axis=0
    )
    acat = jnp.tile(acat, (4, 1))
    BLK = 1000
    xh, a4 = pl.pallas_call(
        _encode_body,
        grid=(N // BLK,),
        in_specs=[
            pl.BlockSpec((BLK, D_FEAT), lambda i: (i, 0)),
            pl.BlockSpec((D_FEAT, HID), lambda i: (0, 0)),
            pl.BlockSpec((8, HID), lambda i: (0, 0)),
            pl.BlockSpec((HID, HEADS * HID), lambda i: (0, 0)),
            pl.BlockSpec((8, HEADS * HID), lambda i: (0, 0)),
        ],
        out_specs=[
            pl.BlockSpec((BLK, HEADS * HID), lambda i: (i, 0)),
            pl.BlockSpec((BLK, 8), lambda i: (i, 0)),
        ],
        out_shape=[
            jax.ShapeDtypeStruct((N, HEADS * HID), jnp.float32),
            jax.ShapeDtypeStruct((N, 8), jnp.float32),
        ],
    )(x, W_enc.T, benc2, W_lin.T, acat)
    return xh, a4


def kernel(x, edge_index, pair_index, W_enc, b_enc, W_lin, att_src, att_dst,
           bias_gat, W1, b1, W2, b2):
    xh, a4 = _encode(x, W_enc, b_enc, W_lin, att_src, att_dst)
    a_s = a4[:, 0:2]
    a_d = a4[:, 2:4]
    w_self = a4[:, 4:6]

    src = edge_index[0]
    dst = edge_index[1]
    t = a_s[src] + a_d[dst]
    w = jnp.exp(jnp.maximum(t, 0.2 * t))
    xhh = xh.reshape(N, HEADS, HID)
    msg = xhh[src] * w[:, :, None]
    A = jax.ops.segment_sum(msg, dst, num_segments=N)
    denom = jax.ops.segment_sum(w, dst, num_segments=N)
    A = A + xhh * w_self[:, :, None]
    denom = denom + w_self
    out = (A / denom[:, :, None]).mean(axis=1) + bias_gat
    h2 = jnp.maximum(out, 0.0)
    nrm = jnp.sqrt(jnp.sum(h2 * h2, axis=1, keepdims=True))
    h2 = h2 / jnp.maximum(nrm, 1e-12)

    U = h2 @ W1[:, :HID].T + b1
    V = h2 @ W1[:, HID:].T
    s = jnp.maximum(U[pair_index[0]] + V[pair_index[1]], 0.0)
    scores = s @ W2[0] + b2[0]
    return scores


# speedup vs baseline: 39.2727x; 37.5202x over previous
import functools

import jax
import jax.numpy as jnp
from jax import lax
from jax.experimental import pallas as pl
from jax.experimental.pallas import tpu as pltpu, tpu_sc as plsc

N = 10000
E = 640000
D_FEAT = 128
HID = 32
HEADS = 2
P = 200000

NC = 2
NS = 16
NW = NC * NS

ROWW = 2 * HID + 16
N_AL = 10240
EPT = E // NW
ECB = 160
EK = 2
ESB = ECB // EK

PPAD = 204800
PPT = PPAD // NW
PCB = 640
PK = 8
PSB = PCB // PK


def _encode_body(x_ref, wenc_ref, benc_ref, wlin_ref, acat_ref, xh_ref, a4_ref):
    h = jnp.maximum(
        jnp.dot(x_ref[...], wenc_ref[...], preferred_element_type=jnp.float32)
        + benc_ref[0:1, :],
        0.0,
    )
    xh = jnp.dot(h, wlin_ref[...], preferred_element_type=jnp.float32)
    xh_ref[...] = xh
    ps = xh * acat_ref[0:1, :]
    pd = xh * acat_ref[1:2, :]
    a_s0 = jnp.sum(ps[:, :HID], axis=1, keepdims=True)
    a_s1 = jnp.sum(ps[:, HID:], axis=1, keepdims=True)
    a_d0 = jnp.sum(pd[:, :HID], axis=1, keepdims=True)
    a_d1 = jnp.sum(pd[:, HID:], axis=1, keepdims=True)
    t0 = a_s0 + a_d0
    t1 = a_s1 + a_d1
    w0 = jnp.exp(jnp.maximum(t0, 0.2 * t0))
    w1 = jnp.exp(jnp.maximum(t1, 0.2 * t1))
    a4_ref[...] = jnp.concatenate([a_s0, a_s1, a_d0, a_d1, w0, w1, t0, t1], axis=1)


def _encode(x, W_enc, b_enc, W_lin, att_src, att_dst):
    benc2 = jnp.tile(b_enc[None, :], (8, 1))
    acat = jnp.concatenate(
        [att_src.reshape(1, HEADS * HID), att_dst.reshape(1, HEADS * HID)], axis=0
    )
    acat = jnp.tile(acat, (4, 1))
    BLK = 1000
    return pl.pallas_call(
        _encode_body,
        grid=(N // BLK,),
        in_specs=[
            pl.BlockSpec((BLK, D_FEAT), lambda i: (i, 0)),
            pl.BlockSpec((D_FEAT, HID), lambda i: (0, 0)),
            pl.BlockSpec((8, HID), lambda i: (0, 0)),
            pl.BlockSpec((HID, HEADS * HID), lambda i: (0, 0)),
            pl.BlockSpec((8, HEADS * HID), lambda i: (0, 0)),
        ],
        out_specs=[
            pl.BlockSpec((BLK, HEADS * HID), lambda i: (i, 0)),
            pl.BlockSpec((BLK, 8), lambda i: (i, 0)),
        ],
        out_shape=[
            jax.ShapeDtypeStruct((N, HEADS * HID), jnp.float32),
            jax.ShapeDtypeStruct((N, 8), jnp.float32),
        ],
    )(x, W_enc.T, benc2, W_lin.T, acat)


def _edge_kernel(src, dst, asrc_flat, adst_flat, xh):
    mesh = plsc.VectorSubcoreMesh(
        core_axis_name="c", subcore_axis_name="s", num_cores=NC, num_subcores=NS
    )

    @functools.partial(
        pl.kernel,
        out_type=jax.ShapeDtypeStruct((NC, N_AL, ROWW), jnp.float32),
        mesh=mesh,
        compiler_params=pltpu.CompilerParams(needs_layout_passes=False, use_tc_tiling_on_sc=False),
        scratch_types=[
            pltpu.VMEM((2 * N,), jnp.float32),
            pltpu.VMEM((2 * N,), jnp.float32),
            pltpu.VMEM((EK, ESB), jnp.int32),
            pltpu.VMEM((EK, ESB), jnp.int32),
            pltpu.VMEM((ECB, 2 * HID), jnp.float32),
            pltpu.VMEM((ECB, ROWW), jnp.float32),
            pltpu.VMEM_SHARED((N_AL, ROWW), jnp.float32),
            pltpu.SemaphoreType.DMA,
        ],
    )
    def body(src_hbm, dst_hbm, asrc_hbm, adst_hbm, xh_hbm, parts_hbm,
             asrc_v, adst_v, sidx_v, didx_v, rows_v, msg_v, a_sh, sem):
        c = lax.axis_index("c")
        s = lax.axis_index("s")
        wid = c * NS + s
        lanes = lax.iota(jnp.int32, 16)
        zeros16 = jnp.zeros((16,), jnp.float32)

        pltpu.sync_copy(asrc_hbm, asrc_v)
        pltpu.sync_copy(adst_hbm, adst_v)

        def _zero_row(r, carry):
            for k in range(ROWW // 16):
                msg_v[r, pl.ds(k * 16, 16)] = zeros16
            return carry
        lax.fori_loop(0, ECB, _zero_row, 0)

        rows0 = s * (N_AL // NS)
        for q in range(N_AL // NS // ECB):
            pltpu.sync_copy(msg_v, a_sh.at[pl.ds(rows0 + q * ECB, ECB), :])
        plsc.subcore_barrier()

        ebase = wid * EPT

        def _chunk(ch, carry):
            e0 = ebase + ch * ECB
            for k in range(EK):
                pltpu.sync_copy(src_hbm.at[pl.ds(e0 + k * ESB, ESB)], sidx_v.at[k])
                pltpu.sync_copy(dst_hbm.at[pl.ds(e0 + k * ESB, ESB)], didx_v.at[k])
            cps = [
                pltpu.async_copy(
                    xh_hbm.at[sidx_v.at[k]], rows_v.at[pl.ds(k * ESB, ESB), :], sem
                )
                for k in range(EK)
            ]
            for cp in cps:
                cp.wait()

            def _group2(g, carry2):
                rid = g * 16 + lanes
                blk = g // (ESB // 16)
                off = (g % (ESB // 16)) * 16
                sv = sidx_v[blk, pl.ds(off, 16)]
                dv = didx_v[blk, pl.ds(off, 16)]
                as0 = plsc.load_gather(asrc_v, [2 * sv])
                as1 = plsc.load_gather(asrc_v, [2 * sv + 1])
                ad0 = plsc.load_gather(adst_v, [2 * dv])
                ad1 = plsc.load_gather(adst_v, [2 * dv + 1])
                t0 = as0 + ad0
                t1 = as1 + ad1
                w0 = jnp.exp(jnp.maximum(t0, 0.2 * t0))
                w1 = jnp.exp(jnp.maximum(t1, 0.2 * t1))
                for col in range(HID):
                    c16 = jnp.full((16,), col, jnp.int32)
                    val = plsc.load_gather(rows_v, [rid, c16])
                    plsc.store_scatter(msg_v, [rid, c16], val * w0)
                for col in range(HID, 2 * HID):
                    c16 = jnp.full((16,), col, jnp.int32)
                    val = plsc.load_gather(rows_v, [rid, c16])
                    plsc.store_scatter(msg_v, [rid, c16], val * w1)
                plsc.store_scatter(msg_v, [rid, jnp.full((16,), 2 * HID, jnp.int32)], w0)
                plsc.store_scatter(msg_v, [rid, jnp.full((16,), 2 * HID + 1, jnp.int32)], w1)
                return carry2
            lax.fori_loop(0, ECB // 16, _group2, 0)

            for k in range(EK):
                pltpu.sync_copy(
                    msg_v.at[pl.ds(k * ESB, ESB), :], a_sh.at[didx_v.at[k]], add=True
                )
            return carry
        lax.fori_loop(0, EPT // ECB, _chunk, 0)

        plsc.subcore_barrier()
        pltpu.sync_copy(a_sh.at[pl.ds(rows0, N_AL // NS), :],
                        parts_hbm.at[c, pl.ds(rows0, N_AL // NS), :])

    return body(src, dst, asrc_flat, adst_flat, xh)


def _combine_body(p0_ref, p1_ref, xh_ref, a4_ref, bias_ref, w1at_ref, w1bt_ref,
                  b1_ref, u_ref, v_ref):
    A = p0_ref[...] + p1_ref[...]
    xh = xh_ref[...]
    ws0 = a4_ref[:, 4:5]
    ws1 = a4_ref[:, 5:6]
    num0 = A[:, 0:HID] + ws0 * xh[:, 0:HID]
    num1 = A[:, HID:2 * HID] + ws1 * xh[:, HID:2 * HID]
    den0 = A[:, 2 * HID:2 * HID + 1] + ws0
    den1 = A[:, 2 * HID + 1:2 * HID + 2] + ws1
    g = 0.5 * (num0 / den0 + num1 / den1) + bias_ref[0:1, :]
    g = jnp.maximum(g, 0.0)
    ss = jnp.sum(g * g, axis=1, keepdims=True)
    g = g / jnp.maximum(jnp.sqrt(ss), 1e-12)
    u_ref[...] = (
        jnp.dot(g, w1at_ref[...], preferred_element_type=jnp.float32)
        + b1_ref[0:1, :]
    )
    v_ref[...] = jnp.dot(g, w1bt_ref[...], preferred_element_type=jnp.float32)


def _combine(p0, p1, xh, a4, bias_gat, W1, b1):
    bias2 = jnp.tile(bias_gat[None, :], (8, 1))
    b12 = jnp.tile(b1[None, :], (8, 1))
    w1at = W1[:, :HID].T
    w1bt = W1[:, HID:].T
    BLK = 1000
    return pl.pallas_call(
        _combine_body,
        grid=(N // BLK,),
        in_specs=[
            pl.BlockSpec((BLK, ROWW), lambda i: (i, 0)),
            pl.BlockSpec((BLK, ROWW), lambda i: (i, 0)),
            pl.BlockSpec((BLK, HEADS * HID), lambda i: (i, 0)),
            pl.BlockSpec((BLK, 8), lambda i: (i, 0)),
            pl.BlockSpec((8, HID), lambda i: (0, 0)),
            pl.BlockSpec((HID, HID), lambda i: (0, 0)),
            pl.BlockSpec((HID, HID), lambda i: (0, 0)),
            pl.BlockSpec((8, HID), lambda i: (0, 0)),
        ],
        out_specs=[
            pl.BlockSpec((BLK, HID), lambda i: (i, 0)),
            pl.BlockSpec((BLK, HID), lambda i: (i, 0)),
        ],
        out_shape=[
            jax.ShapeDtypeStruct((N, HID), jnp.float32),
            jax.ShapeDtypeStruct((N, HID), jnp.float32),
        ],
    )(p0, p1, xh, a4, bias2, w1at, w1bt, b12)


def _pair_kernel(p0_idx, p1_idx, U, V, w2rep, b2rep):
    mesh = plsc.VectorSubcoreMesh(
        core_axis_name="c", subcore_axis_name="s", num_cores=NC, num_subcores=NS
    )

    @functools.partial(
        pl.kernel,
        out_type=jax.ShapeDtypeStruct((PPAD,), jnp.float32),
        mesh=mesh,
        compiler_params=pltpu.CompilerParams(needs_layout_passes=False, use_tc_tiling_on_sc=False),
        scratch_types=[
            pltpu.VMEM((PK, PSB), jnp.int32),
            pltpu.VMEM((PK, PSB), jnp.int32),
            pltpu.VMEM((PCB, HID), jnp.float32),
            pltpu.VMEM((PCB, HID), jnp.float32),
            pltpu.VMEM((HID, 16), jnp.float32),
            pltpu.VMEM((16,), jnp.float32),
            pltpu.VMEM((PCB,), jnp.float32),
            pltpu.SemaphoreType.DMA,
        ],
    )
    def body(p0_hbm, p1_hbm, u_hbm, v_hbm, w2_hbm, b2_hbm, scores_hbm,
             i0_v, i1_v, u_v, v_v, w2_v, b2_v, out_v, sem):
        c = lax.axis_index("c")
        s = lax.axis_index("s")
        wid = c * NS + s
        lanes = lax.iota(jnp.int32, 16)
        pltpu.sync_copy(w2_hbm, w2_v)
        pltpu.sync_copy(b2_hbm, b2_v)
        pbase = wid * PPT

        def _chunk(ch, carry):
            q0 = pbase + ch * PCB
            for k in range(PK):
                pltpu.sync_copy(p0_hbm.at[pl.ds(q0 + k * PSB, PSB)], i0_v.at[k])
                pltpu.sync_copy(p1_hbm.at[pl.ds(q0 + k * PSB, PSB)], i1_v.at[k])
            cps = [
                pltpu.async_copy(u_hbm.at[i0_v.at[k]],
                                 u_v.at[pl.ds(k * PSB, PSB), :], sem)
                for k in range(PK)
            ] + [
                pltpu.async_copy(v_hbm.at[i1_v.at[k]],
                                 v_v.at[pl.ds(k * PSB, PSB), :], sem)
                for k in range(PK)
            ]
            for cp in cps:
                cp.wait()

            def _group(g, carry2):
                rid = g * 16 + lanes
                acc = b2_v[...]
                for j in range(HID):
                    j16 = jnp.full((16,), j, jnp.int32)
                    u = plsc.load_gather(u_v, [rid, j16])
                    v = plsc.load_gather(v_v, [rid, j16])
                    acc = acc + jnp.maximum(u + v, 0.0) * w2_v[j, :]
                out_v[pl.ds(g * 16, 16)] = acc
                return carry2
            lax.fori_loop(0, PCB // 16, _group, 0)
            pltpu.sync_copy(out_v, scores_hbm.at[pl.ds(q0, PCB)])
            return carry
        lax.fori_loop(0, PPT // PCB, _chunk, 0)

    return body(p0_idx, p1_idx, U, V, w2rep, b2rep)


def kernel(x, edge_index, pair_index, W_enc, b_enc, W_lin, att_src, att_dst,
           bias_gat, W1, b1, W2, b2):
    xh, a4 = _encode(x, W_enc, b_enc, W_lin, att_src, att_dst)
    asrc_flat = a4[:, 0:2].reshape(-1)
    adst_flat = a4[:, 2:4].reshape(-1)

    parts = _edge_kernel(edge_index[0], edge_index[1], asrc_flat, adst_flat, xh)
    U, V = _combine(parts[0, :N], parts[1, :N], xh, a4, bias_gat, W1, b1)

    npad = PPAD - P
    p0 = jnp.concatenate([pair_index[0], jnp.zeros((npad,), jnp.int32)])
    p1 = jnp.concatenate([pair_index[1], jnp.zeros((npad,), jnp.int32)])
    w2rep = jnp.tile(W2[0][:, None], (1, 16))
    b2rep = jnp.tile(b2, 16)
    scores = _pair_kernel(p0, p1, U, V, w2rep, b2rep)
    return scores[:P]
